# baseline (speedup 1.0000x reference)
import jax
import jax.numpy as jnp
from jax import lax
from jax.experimental import pallas as pl
from jax.experimental.pallas import tpu as pltpu
from jax.experimental.pallas import tpu_sc as plsc

F32 = jnp.float32

_NC = 2
_NS = 16
_NW = _NC * _NS
_C = 80
_T = 1024


def _leaky(x):
    return jnp.maximum(x, 0.01 * x)


def _wp_body(w_ref, p_ref, o_ref):
    col = lax.broadcasted_iota(jnp.int32, (1, 128), 1)
    vals = jnp.power(w_ref[:, :], p_ref[:, :])
    o_ref[:, :] = jnp.where(col < 3, vals, 0.0)


def _wp_call(elem_weights, pows_pad):
    n = elem_weights.shape[0]
    return pl.pallas_call(
        _wp_body,
        out_shape=jax.ShapeDtypeStruct((n, 128), F32),
    )(elem_weights, pows_pad)


def _gth_body(ein, wp, sidx, nidx, self_o, nbr_o, wp_o, *scr):
    per_w = sidx.shape[0] // _NW
    wid = lax.axis_index("s") * _NC + lax.axis_index("c")
    base = wid * per_w
    nch = per_w // _C
    npair = nch // 2
    slots = (scr[0:5] + (scr[10:13], scr[16:19]),
             scr[5:10] + (scr[13:16], scr[19:22]))

    def load_idx(off, sl):
        pltpu.sync_copy(sidx.at[pl.ds(off, _C)], sl[0])
        pltpu.sync_copy(nidx.at[pl.ds(off, _C)], sl[1])

    def fire_gathers(sl):
        return (pltpu.async_copy(ein.at[sl[0]], sl[2], sl[5][0]),
                pltpu.async_copy(ein.at[sl[1]], sl[3], sl[5][1]),
                pltpu.async_copy(wp.at[sl[1]], sl[4], sl[5][2]))

    def fire_writes(off, sl):
        pltpu.async_copy(sl[2], self_o.at[pl.ds(off, _C)], sl[6][0])
        pltpu.async_copy(sl[3], nbr_o.at[pl.ds(off, _C)], sl[6][1])
        pltpu.async_copy(sl[4], wp_o.at[pl.ds(off, _C)], sl[6][2])

    def wait_writes(off, sl):
        pltpu.make_async_copy(sl[2], self_o.at[pl.ds(off, _C)], sl[6][0]).wait()
        pltpu.make_async_copy(sl[3], nbr_o.at[pl.ds(off, _C)], sl[6][1]).wait()
        pltpu.make_async_copy(sl[4], wp_o.at[pl.ds(off, _C)], sl[6][2]).wait()

    @pl.loop(0, npair)
    def _pair(g):
        off0 = base + (2 * g) * _C
        off1 = off0 + _C

        @pl.when(g > 0)
        def _drain():
            wait_writes(off0, slots[0])
            wait_writes(off1, slots[1])

        load_idx(off0, slots[0])
        cps0 = fire_gathers(slots[0])
        load_idx(off1, slots[1])
        cps1 = fire_gathers(slots[1])
        for cp in cps0:
            cp.wait()
        fire_writes(off0, slots[0])
        for cp in cps1:
            cp.wait()
        fire_writes(off1, slots[1])

    wait_writes(base, slots[0])
    wait_writes(base, slots[1])

    @pl.loop(npair * 2, nch)
    def _tail(i):
        off = base + i * _C
        load_idx(off, slots[0])
        for cp in fire_gathers(slots[0]):
            cp.wait()
        fire_writes(off, slots[0])
        wait_writes(off, slots[0])


def _gth_call(elem_in_fea, wp, self_idx, nbr_idx):
    m = self_idx.shape[0]
    f = elem_in_fea.shape[1]
    mesh = plsc.VectorSubcoreMesh(core_axis_name="c", subcore_axis_name="s")
    slot_bufs = [
        pltpu.VMEM((_C,), jnp.int32),
        pltpu.VMEM((_C,), jnp.int32),
        pltpu.VMEM((_C, f), F32),
        pltpu.VMEM((_C, f), F32),
        pltpu.VMEM((_C, 128), F32),
    ]
    return pl.kernel(
        _gth_body,
        out_type=[
            jax.ShapeDtypeStruct((m, f), F32),
            jax.ShapeDtypeStruct((m, f), F32),
            jax.ShapeDtypeStruct((m, 128), F32),
        ],
        mesh=mesh,
        scratch_types=(slot_bufs + slot_bufs
                       + [pltpu.SemaphoreType.DMA for _ in range(12)]),
    )(elem_in_fea, wp, self_idx, nbr_idx)


def _bf(x):
    return x.astype(jnp.bfloat16)


def _mlp_body(sf_ref, nf_ref, wp_ref, ef_ref, *refs):
    eparams = [(refs[2 * i], refs[2 * i + 1]) for i in range(5)]
    w1cat_ref, b1cat_ref = refs[10], refs[11]
    hrefs = refs[12:24]
    pm_refs = refs[24:27]
    u123_ref = refs[27]
    upk_refs = refs[28:31]

    x = _bf(ef_ref[...])
    for i, (w, b) in enumerate(eparams):
        x = jnp.dot(x, _bf(w[...]), preferred_element_type=F32) + b[...]
        if i < 4:
            x = _leaky(x)
        x = _bf(x)
    fea = jnp.concatenate([_bf(sf_ref[...]), _bf(nf_ref[...]), x], axis=1)

    hid = _bf(_leaky(
        jnp.dot(fea, _bf(w1cat_ref[...]), preferred_element_type=F32)
        + b1cat_ref[...]))

    t = fea.shape[0]
    col = lax.broadcasted_iota(jnp.int32, (t, 128), 1)
    u123 = jnp.zeros((t, 128), F32)
    for h in range(3):
        gw2, gb2, mw2, mb2 = hrefs[4 * h:4 * h + 4]
        hg = hid[:, 512 * h:512 * h + 256]
        hm = hid[:, 512 * h + 256:512 * h + 512]
        g = jnp.dot(hg, _bf(gw2[...]), preferred_element_type=F32) + gb2[...]
        u = wp_ref[:, h:h + 1] * jnp.exp(g)
        msg = jnp.dot(hm, _bf(mw2[...]), preferred_element_type=F32) + mb2[...]
        pm_refs[h][...] = u * msg
        u123 = jnp.where(col == h, u, u123)
        upk_refs[h][...] = u.reshape(t // 128, 128)
    u123_ref[...] = u123


def _mlp_call(selfF, nbrF, wpg, edge_fea, flat_params):
    m, f = selfF.shape
    grid = (m + _T - 1) // _T

    def full(a):
        nd = a.ndim
        return pl.BlockSpec(a.shape, lambda i, _nd=nd: (0,) * _nd)

    data = [selfF, nbrF, wpg, edge_fea]
    in_specs = [pl.BlockSpec((_T, a.shape[1]), lambda i: (i, 0)) for a in data]
    in_specs += [full(p) for p in flat_params]
    out_shape = [jax.ShapeDtypeStruct((m, 128), F32) for _ in range(4)]
    out_specs = [pl.BlockSpec((_T, 128), lambda i: (i, 0)) for _ in range(4)]
    out_shape += [jax.ShapeDtypeStruct((m // 128, 128), F32) for _ in range(3)]
    out_specs += [pl.BlockSpec((_T // 128, 128), lambda i: (i, 0)) for _ in range(3)]
    return pl.pallas_call(
        _mlp_body,
        grid=(grid,),
        in_specs=in_specs,
        out_specs=out_specs,
        out_shape=out_shape,
    )(*data, *flat_params)


def _sct_body(rows_hbm, sidx, prev, part_o, acc_sh,
              idx0, rows0, idx1, rows1, ls0, ls1, ss0, ss1, semz):
    n = prev.shape[1]
    per_w = sidx.shape[0] // _NW
    c = lax.axis_index("c")
    s = lax.axis_index("s")
    wid = s * _NC + c
    rows_per_sub = n // _NS
    r0 = s * rows_per_sub
    pltpu.async_copy(prev.at[c, pl.ds(r0, rows_per_sub)],
                     acc_sh.at[pl.ds(r0, rows_per_sub)], semz).wait()
    plsc.subcore_barrier()
    base = wid * per_w
    nch = per_w // _C
    npair = nch // 2
    slots = ((idx0, rows0, ls0, ss0), (idx1, rows1, ls1, ss1))

    def fire_loads(off, sl):
        pltpu.async_copy(sidx.at[pl.ds(off, _C)], sl[0], sl[2])
        pltpu.async_copy(rows_hbm.at[pl.ds(off, _C)], sl[1], sl[2])

    def wait_loads(off, sl):
        pltpu.make_async_copy(sidx.at[pl.ds(off, _C)], sl[0], sl[2]).wait()
        pltpu.make_async_copy(rows_hbm.at[pl.ds(off, _C)], sl[1], sl[2]).wait()

    def fire_scatter(sl):
        pltpu.async_copy(sl[1], acc_sh.at[sl[0]], sl[3], add=True)

    def wait_scatter(sl):
        pltpu.make_async_copy(sl[1], acc_sh.at[sl[0]], sl[3]).wait()

    @pl.loop(0, npair)
    def _pair(g):
        off0 = base + (2 * g) * _C
        off1 = off0 + _C

        @pl.when(g > 0)
        def _drain():
            wait_scatter(slots[0])
            wait_scatter(slots[1])

        fire_loads(off0, slots[0])
        fire_loads(off1, slots[1])
        wait_loads(off0, slots[0])
        fire_scatter(slots[0])
        wait_loads(off1, slots[1])
        fire_scatter(slots[1])

    wait_scatter(slots[0])
    wait_scatter(slots[1])

    @pl.loop(npair * 2, nch)
    def _tail(i):
        off = base + i * _C
        fire_loads(off, slots[0])
        wait_loads(off, slots[0])
        fire_scatter(slots[0])
        wait_scatter(slots[0])

    plsc.subcore_barrier()
    pltpu.sync_copy(acc_sh.at[pl.ds(r0, rows_per_sub)],
                    part_o.at[c, pl.ds(r0, rows_per_sub)])


def _sct_call(rows, self_idx, prev):
    n = prev.shape[1]
    mesh = plsc.VectorSubcoreMesh(core_axis_name="c", subcore_axis_name="s")
    return pl.kernel(
        _sct_body,
        out_type=jax.ShapeDtypeStruct((_NC, n, 128), F32),
        mesh=mesh,
        scratch_types=[
            pltpu.VMEM_SHARED((n, 128), F32),
            pltpu.VMEM((_C,), jnp.int32),
            pltpu.VMEM((_C, 128), F32),
            pltpu.VMEM((_C,), jnp.int32),
            pltpu.VMEM((_C, 128), F32),
            pltpu.SemaphoreType.DMA,
            pltpu.SemaphoreType.DMA,
            pltpu.SemaphoreType.DMA,
            pltpu.SemaphoreType.DMA,
            pltpu.SemaphoreType.DMA,
        ],
    )(rows, self_idx, prev)


def _fin_body(p0_ref, p1_ref, p2_ref, pu_ref, ein_ref, out_ref,
              gpk0_ref, gpk1_ref, gpk2_ref):
    gs = pu_ref[0] + pu_ref[1]
    rblk = gs.shape[0]
    acc = jnp.zeros_like(ein_ref[...])
    for h, (p_ref, gpk_ref) in enumerate(
            zip((p0_ref, p1_ref, p2_ref), (gpk0_ref, gpk1_ref, gpk2_ref))):
        num = p_ref[0] + p_ref[1]
        gcol = gs[:, h:h + 1]
        acc = acc + num / (gcol + 1e-10)
        gpk_ref[...] = gcol.reshape(rblk // 128, 128)
    out_ref[...] = acc * (1.0 / 3.0) + ein_ref[...]


def _fin_call(p0, p1, p2, pu, elem_in_fea):
    n, f = elem_in_fea.shape
    rblk = 2048
    grid = n // rblk
    pspec = pl.BlockSpec((_NC, rblk, 128), lambda i: (0, i, 0))
    espec = pl.BlockSpec((rblk, f), lambda i: (i, 0))
    gpk_spec = pl.BlockSpec((rblk // 128, 128), lambda i: (i, 0))
    gpk_shape = jax.ShapeDtypeStruct((n // 128, 128), F32)
    return pl.pallas_call(
        _fin_body,
        grid=(grid,),
        in_specs=[pspec, pspec, pspec, pspec, espec],
        out_specs=[espec, gpk_spec, gpk_spec, gpk_spec],
        out_shape=[jax.ShapeDtypeStruct((n, f), F32),
                   gpk_shape, gpk_shape, gpk_shape],
    )(p0, p1, p2, pu, elem_in_fea)


def _div_body(u0, u1, u2, gpk0, gpk1, gpk2, sidx, g0_o, g1_o, g2_o,
              idx_v, ub0, ub1, ub2, gt0, gt1, gt2, go0, go1, go2):
    per_w = sidx.shape[0] // _NW
    wid = lax.axis_index("s") * _NC + lax.axis_index("c")
    base = wid * per_w
    u_ins = (u0, u1, u2)
    gtabs = (gt0, gt1, gt2)
    ubufs = (ub0, ub1, ub2)
    gouts = (go0, go1, go2)
    gate_os = (g0_o, g1_o, g2_o)
    for h, gpk in enumerate((gpk0, gpk1, gpk2)):
        pltpu.sync_copy(gpk, gtabs[h])

    @pl.loop(0, per_w // _C)
    def _chunk(i):
        off = base + i * _C
        pltpu.sync_copy(sidx.at[pl.ds(off, _C)], idx_v)
        for h in range(3):
            pltpu.sync_copy(u_ins[h].at[pl.ds(off, _C)], ubufs[h])

        @pl.loop(0, _C // 16)
        def _grp(j):
            idxg = idx_v[pl.ds(j * 16, 16)]
            row = lax.shift_right_logical(idxg, 7)
            lane = lax.bitwise_and(idxg, 127)
            for h in range(3):
                gs = plsc.load_gather(gtabs[h], [row, lane])
                uv = ubufs[h][pl.ds(j * 16, 16)]
                gouts[h][pl.ds(j * 16, 16)] = uv / (gs + 1e-10)
        for h in range(3):
            pltpu.sync_copy(gouts[h], gate_os[h].at[pl.ds(off, _C)])


def _div_call(u_flats, gs_pks, self_idx):
    m = self_idx.shape[0]
    mesh = plsc.VectorSubcoreMesh(core_axis_name="c", subcore_axis_name="s")
    return pl.kernel(
        _div_body,
        out_type=[jax.ShapeDtypeStruct((m,), F32) for _ in range(3)],
        compiler_params=pltpu.CompilerParams(needs_layout_passes=False),
        mesh=mesh,
        scratch_types=(
            [pltpu.VMEM((_C,), jnp.int32)]
            + [pltpu.VMEM((_C,), F32) for _ in range(3)]
            + [pltpu.VMEM(gs_pks[0].shape, F32) for _ in range(3)]
            + [pltpu.VMEM((_C,), F32) for _ in range(3)]
        ),
    )(*u_flats, *gs_pks, self_idx)


def kernel(elem_weights, elem_in_fea, edge_fea, self_fea_idx, nbr_fea_idx,
           edge_params, gate_params, msg_params, pows):
    n = elem_in_fea.shape[0]
    sidx = self_fea_idx.astype(jnp.int32)
    nidx = nbr_fea_idx.astype(jnp.int32)

    pows_pad = jnp.zeros((1, 128), F32).at[0, :3].set(pows.astype(F32))
    wp = _wp_call(elem_weights.astype(F32), pows_pad)

    m = sidx.shape[0]
    unit = _NW * _C
    nslices = 8
    per = (m // unit) // nslices
    lens = [per * unit] * (nslices - 1)
    lens.append(m - sum(lens))
    bounds = []
    lo = 0
    for ln in lens:
        bounds.append((lo, ln))
        lo += ln

    flat_params = []
    for (w, b) in edge_params:
        flat_params += [w, b.reshape(1, -1)]
    w1cat = jnp.concatenate(
        [m for h in range(3)
         for m in (gate_params[h][0][0], msg_params[h][0][0])], axis=1)
    b1cat = jnp.concatenate(
        [m for h in range(3)
         for m in (gate_params[h][0][1], msg_params[h][0][1])])
    flat_params += [w1cat, b1cat.reshape(1, -1)]
    for h in range(3):
        gw2, gb2 = gate_params[h][1]
        mw2, mb2 = msg_params[h][1]
        flat_params += [gw2, gb2.reshape(1, -1), mw2, mb2.reshape(1, -1)]
    npad = ((n + 2047) // 2048) * 2048
    parts = [jnp.zeros((_NC, npad, 128), F32)] * 4
    upks = [[], [], []]
    for (lo, ln) in bounds:
        ssl = lax.dynamic_slice_in_dim(sidx, lo, ln)
        nsl = lax.dynamic_slice_in_dim(nidx, lo, ln)
        selfF, nbrF, wpg = _gth_call(elem_in_fea, wp, ssl, nsl)
        efsl = lax.dynamic_slice_in_dim(edge_fea, lo, ln)
        pm0, pm1, pm2, u123, upk0, upk1, upk2 = _mlp_call(
            selfF, nbrF, wpg, efsl, flat_params)
        for h, rows in enumerate((pm0, pm1, pm2, u123)):
            parts[h] = _sct_call(rows, ssl, parts[h])
        for h, u in enumerate((upk0, upk1, upk2)):
            upks[h].append(u)

    ein_pad = jnp.zeros((npad, elem_in_fea.shape[1]), F32).at[:n].set(elem_in_fea)
    out_pad, gpk0, gpk1, gpk2 = _fin_call(parts[0], parts[1], parts[2],
                                          parts[3], ein_pad)
    u_flats = [jnp.concatenate([u.reshape(-1) for u in us]) for us in upks]
    g0, g1, g2 = _div_call(u_flats, (gpk0, gpk1, gpk2), sidx)
    return out_pad[:n], jnp.stack([g0, g1, g2])[:, :, None]


# speedup vs baseline: 1.0992x; 1.0992x over previous
import jax
import jax.numpy as jnp
from jax import lax
from jax.experimental import pallas as pl
from jax.experimental.pallas import tpu as pltpu
from jax.experimental.pallas import tpu_sc as plsc

F32 = jnp.float32

_NC = 2
_NS = 16
_NW = _NC * _NS
_C = 80
_T = 1024


def _leaky(x):
    return jnp.maximum(x, 0.01 * x)


def _wp_body(w_ref, p_ref, o_ref):
    col = lax.broadcasted_iota(jnp.int32, (1, 128), 1)
    vals = jnp.power(w_ref[:, :], p_ref[:, :])
    o_ref[:, :] = jnp.where(col < 3, vals, 0.0)


def _wp_call(elem_weights, pows_pad):
    n = elem_weights.shape[0]
    return pl.pallas_call(
        _wp_body,
        out_shape=jax.ShapeDtypeStruct((n, 128), F32),
    )(elem_weights, pows_pad)


def _gth_body(ein, wp, sidx, nidx, self_o, nbr_o, wp_o, *scr):
    per_w = sidx.shape[0] // _NW
    wid = lax.axis_index("s") * _NC + lax.axis_index("c")
    base = wid * per_w
    nch = per_w // _C
    npair = nch // 2
    slots = (scr[0:5] + (scr[10:13], scr[16:19]),
             scr[5:10] + (scr[13:16], scr[19:22]))

    def load_idx(off, sl):
        pltpu.sync_copy(sidx.at[pl.ds(off, _C)], sl[0])
        pltpu.sync_copy(nidx.at[pl.ds(off, _C)], sl[1])

    def fire_gathers(sl):
        return (pltpu.async_copy(ein.at[sl[0]], sl[2], sl[5][0]),
                pltpu.async_copy(ein.at[sl[1]], sl[3], sl[5][1]),
                pltpu.async_copy(wp.at[sl[1]], sl[4], sl[5][2]))

    def fire_writes(off, sl):
        pltpu.async_copy(sl[2], self_o.at[pl.ds(off, _C)], sl[6][0])
        pltpu.async_copy(sl[3], nbr_o.at[pl.ds(off, _C)], sl[6][1])
        pltpu.async_copy(sl[4], wp_o.at[pl.ds(off, _C)], sl[6][2])

    def wait_writes(off, sl):
        pltpu.make_async_copy(sl[2], self_o.at[pl.ds(off, _C)], sl[6][0]).wait()
        pltpu.make_async_copy(sl[3], nbr_o.at[pl.ds(off, _C)], sl[6][1]).wait()
        pltpu.make_async_copy(sl[4], wp_o.at[pl.ds(off, _C)], sl[6][2]).wait()

    @pl.loop(0, npair)
    def _pair(g):
        off0 = base + (2 * g) * _C
        off1 = off0 + _C

        @pl.when(g > 0)
        def _drain():
            wait_writes(off0, slots[0])
            wait_writes(off1, slots[1])

        load_idx(off0, slots[0])
        cps0 = fire_gathers(slots[0])
        load_idx(off1, slots[1])
        cps1 = fire_gathers(slots[1])
        for cp in cps0:
            cp.wait()
        fire_writes(off0, slots[0])
        for cp in cps1:
            cp.wait()
        fire_writes(off1, slots[1])

    wait_writes(base, slots[0])
    wait_writes(base, slots[1])

    @pl.loop(npair * 2, nch)
    def _tail(i):
        off = base + i * _C
        load_idx(off, slots[0])
        for cp in fire_gathers(slots[0]):
            cp.wait()
        fire_writes(off, slots[0])
        wait_writes(off, slots[0])


def _gth_call(elem_in_fea, wp, self_idx, nbr_idx):
    m = self_idx.shape[0]
    f = elem_in_fea.shape[1]
    mesh = plsc.VectorSubcoreMesh(core_axis_name="c", subcore_axis_name="s")
    slot_bufs = [
        pltpu.VMEM((_C,), jnp.int32),
        pltpu.VMEM((_C,), jnp.int32),
        pltpu.VMEM((_C, f), F32),
        pltpu.VMEM((_C, f), F32),
        pltpu.VMEM((_C, 128), F32),
    ]
    return pl.kernel(
        _gth_body,
        out_type=[
            jax.ShapeDtypeStruct((m, f), F32),
            jax.ShapeDtypeStruct((m, f), F32),
            jax.ShapeDtypeStruct((m, 128), F32),
        ],
        mesh=mesh,
        scratch_types=(slot_bufs + slot_bufs
                       + [pltpu.SemaphoreType.DMA for _ in range(12)]),
    )(elem_in_fea, wp, self_idx, nbr_idx)


def _bf(x):
    return x.astype(jnp.bfloat16)


def _mlp_body(sf_ref, nf_ref, wp_ref, ef_ref, *refs):
    eparams = [(refs[2 * i], refs[2 * i + 1]) for i in range(5)]
    w1cat_ref, b1cat_ref = refs[10], refs[11]
    hrefs = refs[12:24]
    pm_refs = refs[24:27]
    u123_ref = refs[27]
    upk_refs = refs[28:31]

    x = _bf(ef_ref[...])
    for i, (w, b) in enumerate(eparams):
        x = jnp.dot(x, _bf(w[...]), preferred_element_type=F32) + b[...]
        if i < 4:
            x = _leaky(x)
        x = _bf(x)
    fea = jnp.concatenate([_bf(sf_ref[...]), _bf(nf_ref[...]), x], axis=1)

    hid = _bf(_leaky(
        jnp.dot(fea, _bf(w1cat_ref[...]), preferred_element_type=F32)
        + b1cat_ref[...]))

    t = fea.shape[0]
    col = lax.broadcasted_iota(jnp.int32, (t, 128), 1)
    u123 = jnp.zeros((t, 128), F32)
    for h in range(3):
        gw2, gb2, mw2, mb2 = hrefs[4 * h:4 * h + 4]
        hg = hid[:, 512 * h:512 * h + 256]
        hm = hid[:, 512 * h + 256:512 * h + 512]
        g = jnp.dot(hg, _bf(gw2[...]), preferred_element_type=F32) + gb2[...]
        u = wp_ref[:, h:h + 1] * jnp.exp(g)
        msg = jnp.dot(hm, _bf(mw2[...]), preferred_element_type=F32) + mb2[...]
        pm_refs[h][...] = u * msg
        u123 = jnp.where(col == h, u, u123)
        upk_refs[h][...] = u.reshape(t // 128, 128)
    u123_ref[...] = u123


def _mlp_call(selfF, nbrF, wpg, edge_fea, flat_params):
    m, f = selfF.shape
    grid = (m + _T - 1) // _T

    def full(a):
        nd = a.ndim
        return pl.BlockSpec(a.shape, lambda i, _nd=nd: (0,) * _nd)

    data = [selfF, nbrF, wpg, edge_fea]
    in_specs = [pl.BlockSpec((_T, a.shape[1]), lambda i: (i, 0)) for a in data]
    in_specs += [full(p) for p in flat_params]
    out_shape = [jax.ShapeDtypeStruct((m, 128), F32) for _ in range(4)]
    out_specs = [pl.BlockSpec((_T, 128), lambda i: (i, 0)) for _ in range(4)]
    out_shape += [jax.ShapeDtypeStruct((m // 128, 128), F32) for _ in range(3)]
    out_specs += [pl.BlockSpec((_T // 128, 128), lambda i: (i, 0)) for _ in range(3)]
    return pl.pallas_call(
        _mlp_body,
        grid=(grid,),
        in_specs=in_specs,
        out_specs=out_specs,
        out_shape=out_shape,
    )(*data, *flat_params)


def _sct_body(rows_hbm, sidx, prev, part_o, acc_sh,
              idx0, rows0, idx1, rows1, ls0, ls1, ss0, ss1, semz):
    n = prev.shape[1]
    per_w = sidx.shape[0] // _NW
    c = lax.axis_index("c")
    s = lax.axis_index("s")
    wid = s * _NC + c
    rows_per_sub = n // _NS
    r0 = s * rows_per_sub
    pltpu.async_copy(prev.at[c, pl.ds(r0, rows_per_sub)],
                     acc_sh.at[pl.ds(r0, rows_per_sub)], semz).wait()
    plsc.subcore_barrier()
    base = wid * per_w
    nch = per_w // _C
    npair = nch // 2
    slots = ((idx0, rows0, ls0, ss0), (idx1, rows1, ls1, ss1))

    def fire_loads(off, sl):
        pltpu.async_copy(sidx.at[pl.ds(off, _C)], sl[0], sl[2])
        pltpu.async_copy(rows_hbm.at[pl.ds(off, _C)], sl[1], sl[2])

    def wait_loads(off, sl):
        pltpu.make_async_copy(sidx.at[pl.ds(off, _C)], sl[0], sl[2]).wait()
        pltpu.make_async_copy(rows_hbm.at[pl.ds(off, _C)], sl[1], sl[2]).wait()

    def fire_scatter(sl):
        pltpu.async_copy(sl[1], acc_sh.at[sl[0]], sl[3], add=True)

    def wait_scatter(sl):
        pltpu.make_async_copy(sl[1], acc_sh.at[sl[0]], sl[3]).wait()

    @pl.loop(0, npair)
    def _pair(g):
        off0 = base + (2 * g) * _C
        off1 = off0 + _C

        @pl.when(g > 0)
        def _drain():
            wait_scatter(slots[0])
            wait_scatter(slots[1])

        fire_loads(off0, slots[0])
        fire_loads(off1, slots[1])
        wait_loads(off0, slots[0])
        fire_scatter(slots[0])
        wait_loads(off1, slots[1])
        fire_scatter(slots[1])

    wait_scatter(slots[0])
    wait_scatter(slots[1])

    @pl.loop(npair * 2, nch)
    def _tail(i):
        off = base + i * _C
        fire_loads(off, slots[0])
        wait_loads(off, slots[0])
        fire_scatter(slots[0])
        wait_scatter(slots[0])

    plsc.subcore_barrier()
    pltpu.sync_copy(acc_sh.at[pl.ds(r0, rows_per_sub)],
                    part_o.at[c, pl.ds(r0, rows_per_sub)])


def _sct_call(rows, self_idx, prev):
    n = prev.shape[1]
    mesh = plsc.VectorSubcoreMesh(core_axis_name="c", subcore_axis_name="s")
    return pl.kernel(
        _sct_body,
        out_type=jax.ShapeDtypeStruct((_NC, n, 128), F32),
        mesh=mesh,
        scratch_types=[
            pltpu.VMEM_SHARED((n, 128), F32),
            pltpu.VMEM((_C,), jnp.int32),
            pltpu.VMEM((_C, 128), F32),
            pltpu.VMEM((_C,), jnp.int32),
            pltpu.VMEM((_C, 128), F32),
            pltpu.SemaphoreType.DMA,
            pltpu.SemaphoreType.DMA,
            pltpu.SemaphoreType.DMA,
            pltpu.SemaphoreType.DMA,
            pltpu.SemaphoreType.DMA,
        ],
    )(rows, self_idx, prev)


def _fin_body(p0_ref, p1_ref, p2_ref, pu_ref, ein_ref, out_ref,
              gpk0_ref, gpk1_ref, gpk2_ref):
    gs = pu_ref[0] + pu_ref[1]
    rblk = gs.shape[0]
    acc = jnp.zeros_like(ein_ref[...])
    for h, (p_ref, gpk_ref) in enumerate(
            zip((p0_ref, p1_ref, p2_ref), (gpk0_ref, gpk1_ref, gpk2_ref))):
        num = p_ref[0] + p_ref[1]
        gcol = gs[:, h:h + 1]
        acc = acc + num / (gcol + 1e-10)
        gpk_ref[...] = gcol.reshape(rblk // 128, 128)
    out_ref[...] = acc * (1.0 / 3.0) + ein_ref[...]


def _fin_call(p0, p1, p2, pu, elem_in_fea):
    n, f = elem_in_fea.shape
    rblk = 2048
    grid = n // rblk
    pspec = pl.BlockSpec((_NC, rblk, 128), lambda i: (0, i, 0))
    espec = pl.BlockSpec((rblk, f), lambda i: (i, 0))
    gpk_spec = pl.BlockSpec((rblk // 128, 128), lambda i: (i, 0))
    gpk_shape = jax.ShapeDtypeStruct((n // 128, 128), F32)
    return pl.pallas_call(
        _fin_body,
        grid=(grid,),
        in_specs=[pspec, pspec, pspec, pspec, espec],
        out_specs=[espec, gpk_spec, gpk_spec, gpk_spec],
        out_shape=[jax.ShapeDtypeStruct((n, f), F32),
                   gpk_shape, gpk_shape, gpk_shape],
    )(p0, p1, p2, pu, elem_in_fea)


def _div_body(u0, u1, u2, gpk0, gpk1, gpk2, sidx, g0_o, g1_o, g2_o,
              idx_v, ub0, ub1, ub2, gt0, gt1, gt2, go0, go1, go2):
    per_w = sidx.shape[0] // _NW
    wid = lax.axis_index("s") * _NC + lax.axis_index("c")
    base = wid * per_w
    u_ins = (u0, u1, u2)
    gtabs = (gt0, gt1, gt2)
    ubufs = (ub0, ub1, ub2)
    gouts = (go0, go1, go2)
    gate_os = (g0_o, g1_o, g2_o)
    for h, gpk in enumerate((gpk0, gpk1, gpk2)):
        pltpu.sync_copy(gpk, gtabs[h])

    @pl.loop(0, per_w // _C)
    def _chunk(i):
        off = base + i * _C
        pltpu.sync_copy(sidx.at[pl.ds(off, _C)], idx_v)
        for h in range(3):
            pltpu.sync_copy(u_ins[h].at[pl.ds(off, _C)], ubufs[h])

        @pl.loop(0, _C // 16)
        def _grp(j):
            idxg = idx_v[pl.ds(j * 16, 16)]
            row = lax.shift_right_logical(idxg, 7)
            lane = lax.bitwise_and(idxg, 127)
            for h in range(3):
                gs = plsc.load_gather(gtabs[h], [row, lane])
                uv = ubufs[h][pl.ds(j * 16, 16)]
                gouts[h][pl.ds(j * 16, 16)] = uv / (gs + 1e-10)
        for h in range(3):
            pltpu.sync_copy(gouts[h], gate_os[h].at[pl.ds(off, _C)])


def _div_call(u_flats, gs_pks, self_idx):
    m = self_idx.shape[0]
    mesh = plsc.VectorSubcoreMesh(core_axis_name="c", subcore_axis_name="s")
    return pl.kernel(
        _div_body,
        out_type=[jax.ShapeDtypeStruct((m,), F32) for _ in range(3)],
        compiler_params=pltpu.CompilerParams(needs_layout_passes=False),
        mesh=mesh,
        scratch_types=(
            [pltpu.VMEM((_C,), jnp.int32)]
            + [pltpu.VMEM((_C,), F32) for _ in range(3)]
            + [pltpu.VMEM(gs_pks[0].shape, F32) for _ in range(3)]
            + [pltpu.VMEM((_C,), F32) for _ in range(3)]
        ),
    )(*u_flats, *gs_pks, self_idx)


def kernel(elem_weights, elem_in_fea, edge_fea, self_fea_idx, nbr_fea_idx,
           edge_params, gate_params, msg_params, pows):
    n = elem_in_fea.shape[0]
    sidx = self_fea_idx.astype(jnp.int32)
    nidx = nbr_fea_idx.astype(jnp.int32)

    pows_pad = jnp.zeros((1, 128), F32).at[0, :3].set(pows.astype(F32))
    wp = _wp_call(elem_weights.astype(F32), pows_pad)

    m = sidx.shape[0]
    unit = _NW * _C
    nslices = 4
    per = (m // unit) // nslices
    lens = [per * unit] * (nslices - 1)
    lens.append(m - sum(lens))
    bounds = []
    lo = 0
    for ln in lens:
        bounds.append((lo, ln))
        lo += ln

    flat_params = []
    for (w, b) in edge_params:
        flat_params += [w, b.reshape(1, -1)]
    w1cat = jnp.concatenate(
        [m for h in range(3)
         for m in (gate_params[h][0][0], msg_params[h][0][0])], axis=1)
    b1cat = jnp.concatenate(
        [m for h in range(3)
         for m in (gate_params[h][0][1], msg_params[h][0][1])])
    flat_params += [w1cat, b1cat.reshape(1, -1)]
    for h in range(3):
        gw2, gb2 = gate_params[h][1]
        mw2, mb2 = msg_params[h][1]
        flat_params += [gw2, gb2.reshape(1, -1), mw2, mb2.reshape(1, -1)]
    npad = ((n + 2047) // 2048) * 2048
    parts = [jnp.zeros((_NC, npad, 128), F32)] * 4
    upks = [[], [], []]
    for (lo, ln) in bounds:
        ssl = lax.dynamic_slice_in_dim(sidx, lo, ln)
        nsl = lax.dynamic_slice_in_dim(nidx, lo, ln)
        selfF, nbrF, wpg = _gth_call(elem_in_fea, wp, ssl, nsl)
        efsl = lax.dynamic_slice_in_dim(edge_fea, lo, ln)
        pm0, pm1, pm2, u123, upk0, upk1, upk2 = _mlp_call(
            selfF, nbrF, wpg, efsl, flat_params)
        for h, rows in enumerate((pm0, pm1, pm2, u123)):
            parts[h] = _sct_call(rows, ssl, parts[h])
        for h, u in enumerate((upk0, upk1, upk2)):
            upks[h].append(u)

    ein_pad = jnp.zeros((npad, elem_in_fea.shape[1]), F32).at[:n].set(elem_in_fea)
    out_pad, gpk0, gpk1, gpk2 = _fin_call(parts[0], parts[1], parts[2],
                                          parts[3], ein_pad)
    u_flats = [jnp.concatenate([u.reshape(-1) for u in us]) for us in upks]
    g0, g1, g2 = _div_call(u_flats, (gpk0, gpk1, gpk2), sidx)
    return out_pad[:n], jnp.stack([g0, g1, g2])[:, :, None]


# speedup vs baseline: 1.2016x; 1.0932x over previous
import jax
import jax.numpy as jnp
from jax import lax
from jax.experimental import pallas as pl
from jax.experimental.pallas import tpu as pltpu
from jax.experimental.pallas import tpu_sc as plsc

F32 = jnp.float32

_NC = 2
_NS = 16
_NW = _NC * _NS
_C = 80
_T = 1024


def _leaky(x):
    return jnp.maximum(x, 0.01 * x)


def _wp_body(w_ref, p_ref, o_ref):
    col = lax.broadcasted_iota(jnp.int32, (1, 128), 1)
    vals = jnp.power(w_ref[:, :], p_ref[:, :])
    o_ref[:, :] = jnp.where(col < 3, vals, 0.0)


def _wp_call(elem_weights, pows_pad):
    n = elem_weights.shape[0]
    return pl.pallas_call(
        _wp_body,
        out_shape=jax.ShapeDtypeStruct((n, 128), F32),
    )(elem_weights, pows_pad)


def _gth_body(ein, wp, sidx, nidx, self_o, nbr_o, wp_o, *scr):
    per_w = sidx.shape[0] // _NW
    wid = lax.axis_index("s") * _NC + lax.axis_index("c")
    base = wid * per_w
    nch = per_w // _C
    npair = nch // 2
    slots = (scr[0:5] + (scr[10:13], scr[16:19]),
             scr[5:10] + (scr[13:16], scr[19:22]))

    def load_idx(off, sl):
        pltpu.sync_copy(sidx.at[pl.ds(off, _C)], sl[0])
        pltpu.sync_copy(nidx.at[pl.ds(off, _C)], sl[1])

    def fire_gathers(sl):
        return (pltpu.async_copy(ein.at[sl[0]], sl[2], sl[5][0]),
                pltpu.async_copy(ein.at[sl[1]], sl[3], sl[5][1]),
                pltpu.async_copy(wp.at[sl[1]], sl[4], sl[5][2]))

    def fire_writes(off, sl):
        pltpu.async_copy(sl[2], self_o.at[pl.ds(off, _C)], sl[6][0])
        pltpu.async_copy(sl[3], nbr_o.at[pl.ds(off, _C)], sl[6][1])
        pltpu.async_copy(sl[4], wp_o.at[pl.ds(off, _C)], sl[6][2])

    def wait_writes(off, sl):
        pltpu.make_async_copy(sl[2], self_o.at[pl.ds(off, _C)], sl[6][0]).wait()
        pltpu.make_async_copy(sl[3], nbr_o.at[pl.ds(off, _C)], sl[6][1]).wait()
        pltpu.make_async_copy(sl[4], wp_o.at[pl.ds(off, _C)], sl[6][2]).wait()

    @pl.loop(0, npair)
    def _pair(g):
        off0 = base + (2 * g) * _C
        off1 = off0 + _C

        @pl.when(g > 0)
        def _drain():
            wait_writes(off0, slots[0])
            wait_writes(off1, slots[1])

        load_idx(off0, slots[0])
        cps0 = fire_gathers(slots[0])
        load_idx(off1, slots[1])
        cps1 = fire_gathers(slots[1])
        for cp in cps0:
            cp.wait()
        fire_writes(off0, slots[0])
        for cp in cps1:
            cp.wait()
        fire_writes(off1, slots[1])

    wait_writes(base, slots[0])
    wait_writes(base, slots[1])

    @pl.loop(npair * 2, nch)
    def _tail(i):
        off = base + i * _C
        load_idx(off, slots[0])
        for cp in fire_gathers(slots[0]):
            cp.wait()
        fire_writes(off, slots[0])
        wait_writes(off, slots[0])


def _gth_call(elem_in_fea, wp, self_idx, nbr_idx):
    m = self_idx.shape[0]
    f = elem_in_fea.shape[1]
    mesh = plsc.VectorSubcoreMesh(core_axis_name="c", subcore_axis_name="s")
    slot_bufs = [
        pltpu.VMEM((_C,), jnp.int32),
        pltpu.VMEM((_C,), jnp.int32),
        pltpu.VMEM((_C, f), F32),
        pltpu.VMEM((_C, f), F32),
        pltpu.VMEM((_C, 128), F32),
    ]
    return pl.kernel(
        _gth_body,
        out_type=[
            jax.ShapeDtypeStruct((m, f), F32),
            jax.ShapeDtypeStruct((m, f), F32),
            jax.ShapeDtypeStruct((m, 128), F32),
        ],
        mesh=mesh,
        scratch_types=(slot_bufs + slot_bufs
                       + [pltpu.SemaphoreType.DMA for _ in range(12)]),
    )(elem_in_fea, wp, self_idx, nbr_idx)


def _bf(x):
    return x.astype(jnp.bfloat16)


def _mlp_body(sf_ref, nf_ref, wp_ref, ef_ref, *refs):
    eparams = [(refs[2 * i], refs[2 * i + 1]) for i in range(5)]
    w1cat_ref, b1cat_ref = refs[10], refs[11]
    hrefs = refs[12:24]
    pm_refs = refs[24:27]
    u123_ref = refs[27]
    upk_refs = refs[28:31]

    x = _bf(ef_ref[...])
    for i, (w, b) in enumerate(eparams):
        x = jnp.dot(x, _bf(w[...]), preferred_element_type=F32) + b[...]
        if i < 4:
            x = _leaky(x)
        x = _bf(x)
    fea = jnp.concatenate([_bf(sf_ref[...]), _bf(nf_ref[...]), x], axis=1)

    hid = _bf(_leaky(
        jnp.dot(fea, _bf(w1cat_ref[...]), preferred_element_type=F32)
        + b1cat_ref[...]))

    t = fea.shape[0]
    col = lax.broadcasted_iota(jnp.int32, (t, 128), 1)
    u123 = jnp.zeros((t, 128), F32)
    for h in range(3):
        gw2, gb2, mw2, mb2 = hrefs[4 * h:4 * h + 4]
        hg = hid[:, 512 * h:512 * h + 256]
        hm = hid[:, 512 * h + 256:512 * h + 512]
        g = jnp.dot(hg, _bf(gw2[...]), preferred_element_type=F32) + gb2[...]
        u = wp_ref[:, h:h + 1] * jnp.exp(g)
        msg = jnp.dot(hm, _bf(mw2[...]), preferred_element_type=F32) + mb2[...]
        pm_refs[h][...] = u * msg
        u123 = jnp.where(col == h, u, u123)
        upk_refs[h][...] = u.reshape(t // 128, 128)
    u123_ref[...] = u123


def _mlp_call(selfF, nbrF, wpg, edge_fea, flat_params):
    m, f = selfF.shape
    grid = (m + _T - 1) // _T

    def full(a):
        nd = a.ndim
        return pl.BlockSpec(a.shape, lambda i, _nd=nd: (0,) * _nd)

    data = [selfF, nbrF, wpg, edge_fea]
    in_specs = [pl.BlockSpec((_T, a.shape[1]), lambda i: (i, 0)) for a in data]
    in_specs += [full(p) for p in flat_params]
    out_shape = [jax.ShapeDtypeStruct((m, 128), F32) for _ in range(4)]
    out_specs = [pl.BlockSpec((_T, 128), lambda i: (i, 0)) for _ in range(4)]
    out_shape += [jax.ShapeDtypeStruct((m // 128, 128), F32) for _ in range(3)]
    out_specs += [pl.BlockSpec((_T // 128, 128), lambda i: (i, 0)) for _ in range(3)]
    return pl.pallas_call(
        _mlp_body,
        grid=(grid,),
        in_specs=in_specs,
        out_specs=out_specs,
        out_shape=out_shape,
    )(*data, *flat_params)


def _sct_body(rows_hbm, sidx, prev, part_o, acc_sh,
              idx0, rows0, idx1, rows1, ls0, ls1, ss0, ss1, semz):
    n = prev.shape[1]
    per_w = sidx.shape[0] // _NW
    c = lax.axis_index("c")
    s = lax.axis_index("s")
    wid = s * _NC + c
    rows_per_sub = n // _NS
    r0 = s * rows_per_sub
    pltpu.async_copy(prev.at[c, pl.ds(r0, rows_per_sub)],
                     acc_sh.at[pl.ds(r0, rows_per_sub)], semz).wait()
    plsc.subcore_barrier()
    base = wid * per_w
    nch = per_w // _C
    npair = nch // 2
    slots = ((idx0, rows0, ls0, ss0), (idx1, rows1, ls1, ss1))

    def fire_loads(off, sl):
        pltpu.async_copy(sidx.at[pl.ds(off, _C)], sl[0], sl[2])
        pltpu.async_copy(rows_hbm.at[pl.ds(off, _C)], sl[1], sl[2])

    def wait_loads(off, sl):
        pltpu.make_async_copy(sidx.at[pl.ds(off, _C)], sl[0], sl[2]).wait()
        pltpu.make_async_copy(rows_hbm.at[pl.ds(off, _C)], sl[1], sl[2]).wait()

    def fire_scatter(sl):
        pltpu.async_copy(sl[1], acc_sh.at[sl[0]], sl[3], add=True)

    def wait_scatter(sl):
        pltpu.make_async_copy(sl[1], acc_sh.at[sl[0]], sl[3]).wait()

    @pl.loop(0, npair)
    def _pair(g):
        off0 = base + (2 * g) * _C
        off1 = off0 + _C

        @pl.when(g > 0)
        def _drain():
            wait_scatter(slots[0])
            wait_scatter(slots[1])

        fire_loads(off0, slots[0])
        fire_loads(off1, slots[1])
        wait_loads(off0, slots[0])
        fire_scatter(slots[0])
        wait_loads(off1, slots[1])
        fire_scatter(slots[1])

    wait_scatter(slots[0])
    wait_scatter(slots[1])

    @pl.loop(npair * 2, nch)
    def _tail(i):
        off = base + i * _C
        fire_loads(off, slots[0])
        wait_loads(off, slots[0])
        fire_scatter(slots[0])
        wait_scatter(slots[0])

    plsc.subcore_barrier()
    pltpu.sync_copy(acc_sh.at[pl.ds(r0, rows_per_sub)],
                    part_o.at[c, pl.ds(r0, rows_per_sub)])


def _sct_call(rows, self_idx, prev):
    n = prev.shape[1]
    mesh = plsc.VectorSubcoreMesh(core_axis_name="c", subcore_axis_name="s")
    return pl.kernel(
        _sct_body,
        out_type=jax.ShapeDtypeStruct((_NC, n, 128), F32),
        mesh=mesh,
        scratch_types=[
            pltpu.VMEM_SHARED((n, 128), F32),
            pltpu.VMEM((_C,), jnp.int32),
            pltpu.VMEM((_C, 128), F32),
            pltpu.VMEM((_C,), jnp.int32),
            pltpu.VMEM((_C, 128), F32),
            pltpu.SemaphoreType.DMA,
            pltpu.SemaphoreType.DMA,
            pltpu.SemaphoreType.DMA,
            pltpu.SemaphoreType.DMA,
            pltpu.SemaphoreType.DMA,
        ],
    )(rows, self_idx, prev)


def _fin_body(p0_ref, p1_ref, p2_ref, pu_ref, ein_ref, out_ref,
              gpk0_ref, gpk1_ref, gpk2_ref):
    gs = pu_ref[0] + pu_ref[1]
    rblk = gs.shape[0]
    acc = jnp.zeros_like(ein_ref[...])
    for h, (p_ref, gpk_ref) in enumerate(
            zip((p0_ref, p1_ref, p2_ref), (gpk0_ref, gpk1_ref, gpk2_ref))):
        num = p_ref[0] + p_ref[1]
        gcol = gs[:, h:h + 1]
        acc = acc + num / (gcol + 1e-10)
        gpk_ref[...] = gcol.reshape(rblk // 128, 128)
    out_ref[...] = acc * (1.0 / 3.0) + ein_ref[...]


def _fin_call(p0, p1, p2, pu, elem_in_fea):
    n, f = elem_in_fea.shape
    rblk = 2048
    grid = n // rblk
    pspec = pl.BlockSpec((_NC, rblk, 128), lambda i: (0, i, 0))
    espec = pl.BlockSpec((rblk, f), lambda i: (i, 0))
    gpk_spec = pl.BlockSpec((rblk // 128, 128), lambda i: (i, 0))
    gpk_shape = jax.ShapeDtypeStruct((n // 128, 128), F32)
    return pl.pallas_call(
        _fin_body,
        grid=(grid,),
        in_specs=[pspec, pspec, pspec, pspec, espec],
        out_specs=[espec, gpk_spec, gpk_spec, gpk_spec],
        out_shape=[jax.ShapeDtypeStruct((n, f), F32),
                   gpk_shape, gpk_shape, gpk_shape],
    )(p0, p1, p2, pu, elem_in_fea)


def _div_body(u0, u1, u2, gpk0, gpk1, gpk2, sidx, g0_o, g1_o, g2_o, *scr):
    per_w = sidx.shape[0] // _NW
    wid = lax.axis_index("s") * _NC + lax.axis_index("c")
    base = wid * per_w
    u_ins = (u0, u1, u2)
    gate_os = (g0_o, g1_o, g2_o)
    gtabs = scr[0:3]
    slots = ((scr[3], scr[4:7], scr[7:10], scr[17], scr[19]),
             (scr[10], scr[11:14], scr[14:17], scr[18], scr[20]))
    for h, gpk in enumerate((gpk0, gpk1, gpk2)):
        pltpu.sync_copy(gpk, gtabs[h])
    nch = per_w // _C
    npair = nch // 2

    def fire_loads(off, sl):
        pltpu.async_copy(sidx.at[pl.ds(off, _C)], sl[0], sl[3])
        for h in range(3):
            pltpu.async_copy(u_ins[h].at[pl.ds(off, _C)], sl[1][h], sl[3])

    def wait_loads(off, sl):
        pltpu.make_async_copy(sidx.at[pl.ds(off, _C)], sl[0], sl[3]).wait()
        for h in range(3):
            pltpu.make_async_copy(u_ins[h].at[pl.ds(off, _C)], sl[1][h], sl[3]).wait()

    def compute(sl):
        @pl.loop(0, _C // 16)
        def _grp(j):
            idxg = sl[0][pl.ds(j * 16, 16)]
            row = lax.shift_right_logical(idxg, 7)
            lane = lax.bitwise_and(idxg, 127)
            for h in range(3):
                gs = plsc.load_gather(gtabs[h], [row, lane])
                uv = sl[1][h][pl.ds(j * 16, 16)]
                sl[2][h][pl.ds(j * 16, 16)] = uv / (gs + 1e-10)

    def fire_writes(off, sl):
        for h in range(3):
            pltpu.async_copy(sl[2][h], gate_os[h].at[pl.ds(off, _C)], sl[4])

    def wait_writes(off, sl):
        for h in range(3):
            pltpu.make_async_copy(sl[2][h], gate_os[h].at[pl.ds(off, _C)], sl[4]).wait()

    @pl.loop(0, npair)
    def _pair(g):
        off0 = base + (2 * g) * _C
        off1 = off0 + _C

        @pl.when(g > 0)
        def _drain():
            wait_writes(off0, slots[0])
            wait_writes(off1, slots[1])

        fire_loads(off0, slots[0])
        fire_loads(off1, slots[1])
        wait_loads(off0, slots[0])
        compute(slots[0])
        fire_writes(off0, slots[0])
        wait_loads(off1, slots[1])
        compute(slots[1])
        fire_writes(off1, slots[1])

    wait_writes(base, slots[0])
    wait_writes(base, slots[1])

    @pl.loop(npair * 2, nch)
    def _tail(i):
        off = base + i * _C
        fire_loads(off, slots[0])
        wait_loads(off, slots[0])
        compute(slots[0])
        fire_writes(off, slots[0])
        wait_writes(off, slots[0])


def _div_call(u_flats, gs_pks, self_idx):
    m = self_idx.shape[0]
    mesh = plsc.VectorSubcoreMesh(core_axis_name="c", subcore_axis_name="s")
    slot = ([pltpu.VMEM((_C,), jnp.int32)]
            + [pltpu.VMEM((_C,), F32) for _ in range(6)])
    return pl.kernel(
        _div_body,
        out_type=[jax.ShapeDtypeStruct((m,), F32) for _ in range(3)],
        compiler_params=pltpu.CompilerParams(needs_layout_passes=False),
        mesh=mesh,
        scratch_types=(
            [pltpu.VMEM(gs_pks[0].shape, F32) for _ in range(3)]
            + slot + slot
            + [pltpu.SemaphoreType.DMA for _ in range(4)]
        ),
    )(*u_flats, *gs_pks, self_idx)


def kernel(elem_weights, elem_in_fea, edge_fea, self_fea_idx, nbr_fea_idx,
           edge_params, gate_params, msg_params, pows):
    n = elem_in_fea.shape[0]
    sidx = self_fea_idx.astype(jnp.int32)
    nidx = nbr_fea_idx.astype(jnp.int32)

    pows_pad = jnp.zeros((1, 128), F32).at[0, :3].set(pows.astype(F32))
    wp = _wp_call(elem_weights.astype(F32), pows_pad)

    m = sidx.shape[0]
    unit = _NW * _C
    nslices = 4
    per = (m // unit) // nslices
    lens = [per * unit] * (nslices - 1)
    lens.append(m - sum(lens))
    bounds = []
    lo = 0
    for ln in lens:
        bounds.append((lo, ln))
        lo += ln

    flat_params = []
    for (w, b) in edge_params:
        flat_params += [w, b.reshape(1, -1)]
    w1cat = jnp.concatenate(
        [m for h in range(3)
         for m in (gate_params[h][0][0], msg_params[h][0][0])], axis=1)
    b1cat = jnp.concatenate(
        [m for h in range(3)
         for m in (gate_params[h][0][1], msg_params[h][0][1])])
    flat_params += [w1cat, b1cat.reshape(1, -1)]
    for h in range(3):
        gw2, gb2 = gate_params[h][1]
        mw2, mb2 = msg_params[h][1]
        flat_params += [gw2, gb2.reshape(1, -1), mw2, mb2.reshape(1, -1)]
    npad = ((n + 2047) // 2048) * 2048
    parts = [jnp.zeros((_NC, npad, 128), F32)] * 4
    upks = [[], [], []]
    for (lo, ln) in bounds:
        ssl = lax.dynamic_slice_in_dim(sidx, lo, ln)
        nsl = lax.dynamic_slice_in_dim(nidx, lo, ln)
        selfF, nbrF, wpg = _gth_call(elem_in_fea, wp, ssl, nsl)
        efsl = lax.dynamic_slice_in_dim(edge_fea, lo, ln)
        pm0, pm1, pm2, u123, upk0, upk1, upk2 = _mlp_call(
            selfF, nbrF, wpg, efsl, flat_params)
        for h, rows in enumerate((pm0, pm1, pm2, u123)):
            parts[h] = _sct_call(rows, ssl, parts[h])
        for h, u in enumerate((upk0, upk1, upk2)):
            upks[h].append(u)

    ein_pad = jnp.zeros((npad, elem_in_fea.shape[1]), F32).at[:n].set(elem_in_fea)
    out_pad, gpk0, gpk1, gpk2 = _fin_call(parts[0], parts[1], parts[2],
                                          parts[3], ein_pad)
    u_flats = [jnp.concatenate([u.reshape(-1) for u in us]) for us in upks]
    g0, g1, g2 = _div_call(u_flats, (gpk0, gpk1, gpk2), sidx)
    return out_pad[:n], jnp.stack([g0, g1, g2])[:, :, None]
